# Initial kernel scaffold; baseline (speedup 1.0000x reference)
import jax
import jax.numpy as jnp
from jax.experimental import pallas as pl


def kernel(pos, batch):
    raise NotImplementedError("write your pallas kernel here")


# speedup vs baseline: 14.9368x; 14.9368x over previous
import functools

import jax
import jax.numpy as jnp
from jax import lax
from jax.experimental import pallas as pl
from jax.experimental.pallas import tpu as pltpu
from jax.experimental.pallas import tpu_sc as plsc

_N = 4096
_K = 32
_CUT2 = 4.0
_L = 16

_info = plsc.get_sparse_core_info()
_NC = _info.num_cores
_NS = _info.num_subcores
_NW = _NC * _NS
_RPW = _N // _NW

_mesh = plsc.VectorSubcoreMesh(core_axis_name="c", subcore_axis_name="s")


@functools.partial(
    pl.kernel,
    mesh=_mesh,
    compiler_params=pltpu.CompilerParams(needs_layout_passes=False),
    out_type=[
        jax.ShapeDtypeStruct((_N * _K,), jnp.int32),
        jax.ShapeDtypeStruct((_N * _K,), jnp.float32),
    ],
    scratch_types=[
        pltpu.VMEM((_N,), jnp.float32),
        pltpu.VMEM((_N,), jnp.float32),
        pltpu.VMEM((_N,), jnp.float32),
        pltpu.VMEM((_N,), jnp.int32),
        pltpu.VMEM((_N,), jnp.int32),
        pltpu.VMEM((_N,), jnp.float32),
        pltpu.VMEM((_RPW * _K,), jnp.int32),
        pltpu.VMEM((_RPW * _K,), jnp.float32),
    ],
)
def _sc_radius(x_hbm, y_hbm, z_hbm, ss_hbm, se_hbm, nbr_hbm, sq_hbm,
               xv, yv, zv, ssv, sev, distv, nbrv, sqv):
    wid = lax.axis_index("s") * _NC + lax.axis_index("c")
    base = wid * _RPW

    pltpu.sync_copy(x_hbm, xv)
    pltpu.sync_copy(y_hbm, yv)
    pltpu.sync_copy(z_hbm, zv)
    pltpu.sync_copy(ss_hbm, ssv)
    pltpu.sync_copy(se_hbm, sev)

    lanes = lax.broadcasted_iota(jnp.int32, (_L,), 0)
    lane0 = lanes == 0
    inf = jnp.float32(jnp.inf)
    inf_vec = jnp.full((_L,), inf, jnp.float32)

    def per_node(t, _):
        i = base + t
        iv = jnp.full((_L,), i, jnp.int32)
        sv = plsc.load_gather(ssv, [iv])
        ev = plsc.load_gather(sev, [iv])
        xiv = plsc.load_gather(xv, [iv])
        yiv = plsc.load_gather(yv, [iv])
        ziv = plsc.load_gather(zv, [iv])
        s = sv[0]
        e = ev[0]
        c0 = s // _L
        c1 = (e + (_L - 1)) // _L

        def dchunk(c, carry):
            off = pl.multiple_of(c * _L, _L)
            jj = off + lanes
            dx = xiv - xv[pl.ds(off, _L)]
            dy = yiv - yv[pl.ds(off, _L)]
            dz = ziv - zv[pl.ds(off, _L)]
            sq = dx * dx + dy * dy
            sq = sq + dz * dz
            ok = (jj >= sv) & (jj < ev) & (jj != iv) & (sq < _CUT2)
            distv[pl.ds(off, _L)] = jnp.where(ok, sq, inf)
            return carry

        lax.fori_loop(c0, c1, dchunk, 0)

        for q in range(_K // _L):
            nbrv[pl.ds(t * _K + q * _L, _L)] = iv
            sqv[pl.ds(t * _K + q * _L, _L)] = jnp.zeros((_L,), jnp.float32)

        def sel_cond(carry):
            k, done = carry
            return jnp.logical_and(k < _K, jnp.logical_not(done))

        def sel_body(carry):
            k, _ = carry

            def mchunk(c, mc):
                vmin, varg = mc
                d = distv[pl.ds(pl.multiple_of(c * _L, _L), _L)]
                lt = d < vmin
                return jnp.where(lt, d, vmin), jnp.where(lt, c, varg)

            vmin, varg = lax.fori_loop(
                c0, c1, mchunk,
                (inf_vec, jnp.zeros((_L,), jnp.int32)))
            minval = jnp.min(vmin)
            found = minval < inf

            @pl.when(found)
            def _():
                j = jnp.min(jnp.where(vmin == minval, varg * _L + lanes,
                                      jnp.int32(2**30)))
                jv = jnp.broadcast_to(j, (_L,))
                slotv = jnp.broadcast_to(t * _K + k, (_L,))
                plsc.store_scatter(distv, [jv], inf_vec, mask=lane0)
                plsc.store_scatter(nbrv, [slotv], jv, mask=lane0)
                plsc.store_scatter(sqv, [slotv],
                                   jnp.broadcast_to(minval, (_L,)), mask=lane0)

            return k + jnp.where(found, 1, 0), jnp.logical_not(found)

        lax.while_loop(sel_cond, sel_body, (jnp.int32(0), False))
        return _

    lax.fori_loop(0, _RPW, per_node, 0)

    pltpu.sync_copy(nbrv, nbr_hbm.at[pl.ds(base * _K, _RPW * _K)])
    pltpu.sync_copy(sqv, sq_hbm.at[pl.ds(base * _K, _RPW * _K)])


def _sqrt_body(x_ref, o_ref):
    o_ref[...] = jnp.sqrt(x_ref[...])


_tc_sqrt = pl.pallas_call(
    _sqrt_body,
    out_shape=jax.ShapeDtypeStruct((_N * _K // 1024, 1024), jnp.float32),
)


def kernel(pos, batch):
    pos = pos.astype(jnp.float32)
    batch = batch.astype(jnp.int32)
    x = pos[:, 0]
    y = pos[:, 1]
    z = pos[:, 2]
    ss = jnp.searchsorted(batch, batch, side="left").astype(jnp.int32)
    se = jnp.searchsorted(batch, batch, side="right").astype(jnp.int32)
    nbr, sq = _sc_radius(x, y, z, ss, se)
    w = _tc_sqrt(sq.reshape(_N * _K // 1024, 1024)).reshape(-1)
    col = jnp.broadcast_to(
        jnp.arange(_N, dtype=jnp.int32)[:, None], (_N, _K)).reshape(-1)
    edge_index = jnp.stack([nbr, col], axis=0)
    return edge_index, w


# speedup vs baseline: 45.6917x; 3.0590x over previous
import functools

import jax
import jax.numpy as jnp
from jax import lax
from jax.experimental import pallas as pl
from jax.experimental.pallas import tpu as pltpu
from jax.experimental.pallas import tpu_sc as plsc

_N = 4096
_K = 32
_CUT2 = 4.0
_L = 16

_info = plsc.get_sparse_core_info()
_NC = _info.num_cores
_NS = _info.num_subcores
_NW = _NC * _NS
_RPW = _N // _NW

_mesh = plsc.VectorSubcoreMesh(core_axis_name="c", subcore_axis_name="s")


@functools.partial(
    pl.kernel,
    mesh=_mesh,
    compiler_params=pltpu.CompilerParams(needs_layout_passes=False),
    out_type=[
        jax.ShapeDtypeStruct((_N * _K,), jnp.int32),
        jax.ShapeDtypeStruct((_N * _K,), jnp.float32),
    ],
    scratch_types=[
        pltpu.VMEM((_N,), jnp.float32),
        pltpu.VMEM((_N,), jnp.float32),
        pltpu.VMEM((_N,), jnp.float32),
        pltpu.VMEM((_N,), jnp.int32),
        pltpu.VMEM((_N,), jnp.int32),
        pltpu.VMEM((_N,), jnp.float32),
        pltpu.VMEM((_N,), jnp.int32),
        pltpu.VMEM((_RPW * _K,), jnp.int32),
        pltpu.VMEM((_RPW * _K,), jnp.float32),
    ],
)
def _sc_radius(x_hbm, y_hbm, z_hbm, ss_hbm, se_hbm, nbr_hbm, sq_hbm,
               xv, yv, zv, ssv, sev, cdist, cidx, nbrv, sqv):
    wid = lax.axis_index("s") * _NC + lax.axis_index("c")
    base = wid * _RPW

    pltpu.sync_copy(x_hbm, xv)
    pltpu.sync_copy(y_hbm, yv)
    pltpu.sync_copy(z_hbm, zv)
    pltpu.sync_copy(ss_hbm, ssv)
    pltpu.sync_copy(se_hbm, sev)

    lanes = lax.broadcasted_iota(jnp.int32, (_L,), 0)
    lane0 = lanes == 0
    inf = jnp.float32(jnp.inf)
    inf_vec = jnp.full((_L,), inf, jnp.float32)
    zero_vec = jnp.zeros((_L,), jnp.float32)

    def per_node(t, carry_unused):
        i = base + t
        iv = jnp.full((_L,), i, jnp.int32)
        sv = plsc.load_gather(ssv, [iv])
        ev = plsc.load_gather(sev, [iv])
        xiv = plsc.load_gather(xv, [iv])
        yiv = plsc.load_gather(yv, [iv])
        ziv = plsc.load_gather(zv, [iv])
        s = sv[0]
        e = ev[0]
        c0 = s // _L
        c1 = (e + (_L - 1)) // _L

        def dchunk(c, cnt):
            off = pl.multiple_of(c * _L, _L)
            jj = off + lanes
            dx = xiv - xv[pl.ds(off, _L)]
            dy = yiv - yv[pl.ds(off, _L)]
            dz = ziv - zv[pl.ds(off, _L)]
            sq = dx * dx + dy * dy
            sq = sq + dz * dz
            ok = (jj >= sv) & (jj < ev) & (jj != iv) & (sq < _CUT2)
            cs = plsc.cumsum(jnp.where(ok, 1, 0))
            posn = cnt + cs - 1
            plsc.store_scatter(cdist, [posn], sq, mask=ok)
            plsc.store_scatter(cidx, [posn], jj, mask=ok)
            return cnt + cs[_L - 1]

        ncand = lax.fori_loop(c0, c1, dchunk, jnp.int32(0))
        ncv = jnp.broadcast_to(ncand, (_L,))

        for q in range(_K // _L):
            nbrv[pl.ds(t * _K + q * _L, _L)] = iv
            sqv[pl.ds(t * _K + q * _L, _L)] = zero_vec

        @pl.when(ncand <= _L)
        def _():
            keys = jnp.where(lanes < ncv, cdist[pl.ds(0, _L)], inf)
            vals = cidx[pl.ds(0, _L)]
            ks, vs = plsc.sort_key_val(keys, vals)
            pad = ks == inf
            nbrv[pl.ds(t * _K, _L)] = jnp.where(pad, iv, vs)
            sqv[pl.ds(t * _K, _L)] = jnp.where(pad, zero_vec, ks)

        @pl.when(ncand > _L)
        def _():
            nch = (ncand + (_L - 1)) // _L

            def sel_cond(carry):
                k, done = carry
                return jnp.logical_and(k < _K, jnp.logical_not(done))

            def sel_body(carry):
                k, _d = carry

                def mchunk(c, mc):
                    vmin, varg = mc
                    off = pl.multiple_of(c * _L, _L)
                    d = cdist[pl.ds(off, _L)]
                    d = jnp.where(off + lanes < ncv, d, inf)
                    lt = d < vmin
                    return jnp.where(lt, d, vmin), jnp.where(lt, c, varg)

                vmin, varg = lax.fori_loop(
                    0, nch, mchunk,
                    (inf_vec, jnp.zeros((_L,), jnp.int32)))
                minval = jnp.min(vmin)
                found = minval < inf

                @pl.when(found)
                def _():
                    j = jnp.min(jnp.where(vmin == minval, varg * _L + lanes,
                                          jnp.int32(2**30)))
                    jv = jnp.broadcast_to(j, (_L,))
                    nb = plsc.load_gather(cidx, [jv])
                    slotv = jnp.broadcast_to(t * _K + k, (_L,))
                    plsc.store_scatter(cdist, [jv], inf_vec, mask=lane0)
                    plsc.store_scatter(nbrv, [slotv], nb, mask=lane0)
                    plsc.store_scatter(sqv, [slotv],
                                       jnp.broadcast_to(minval, (_L,)),
                                       mask=lane0)

                return k + jnp.where(found, 1, 0), jnp.logical_not(found)

            lax.while_loop(sel_cond, sel_body, (jnp.int32(0), False))

        return carry_unused

    lax.fori_loop(0, _RPW, per_node, 0)

    pltpu.sync_copy(nbrv, nbr_hbm.at[pl.ds(base * _K, _RPW * _K)])
    pltpu.sync_copy(sqv, sq_hbm.at[pl.ds(base * _K, _RPW * _K)])


def _sqrt_body(x_ref, o_ref):
    o_ref[...] = jnp.sqrt(x_ref[...])


_tc_sqrt = pl.pallas_call(
    _sqrt_body,
    out_shape=jax.ShapeDtypeStruct((_N * _K // 1024, 1024), jnp.float32),
)


def _segment_bounds(batch):
    idx = jnp.arange(_N, dtype=jnp.int32)
    ne = batch[1:] != batch[:-1]
    is_start = jnp.concatenate([jnp.ones((1,), bool), ne])
    is_end = jnp.concatenate([ne, jnp.ones((1,), bool)])
    ss = lax.cummax(jnp.where(is_start, idx, 0))
    se = lax.cummin(jnp.where(is_end, idx + 1, _N)[::-1])[::-1]
    return ss, se


def kernel(pos, batch):
    pos = pos.astype(jnp.float32)
    batch = batch.astype(jnp.int32)
    x = pos[:, 0]
    y = pos[:, 1]
    z = pos[:, 2]
    ss, se = _segment_bounds(batch)
    nbr, sq = _sc_radius(x, y, z, ss, se)
    w = _tc_sqrt(sq.reshape(_N * _K // 1024, 1024)).reshape(-1)
    col = jnp.broadcast_to(
        jnp.arange(_N, dtype=jnp.int32)[:, None], (_N, _K)).reshape(-1)
    edge_index = jnp.stack([nbr, col], axis=0)
    return edge_index, w


# speedup vs baseline: 56.2756x; 1.2316x over previous
import functools

import jax
import jax.numpy as jnp
from jax import lax
from jax.experimental import pallas as pl
from jax.experimental.pallas import tpu as pltpu
from jax.experimental.pallas import tpu_sc as plsc

_N = 4096
_K = 32
_CUT2 = 4.0
_L = 16

_info = plsc.get_sparse_core_info()
_NC = _info.num_cores
_NS = _info.num_subcores
_NW = _NC * _NS
_RPW = _N // _NW

_mesh = plsc.VectorSubcoreMesh(core_axis_name="c", subcore_axis_name="s")


@functools.partial(
    pl.kernel,
    mesh=_mesh,
    compiler_params=pltpu.CompilerParams(needs_layout_passes=False),
    out_type=[
        jax.ShapeDtypeStruct((_NW, _RPW, _K), jnp.int32),
        jax.ShapeDtypeStruct((_NW, _RPW, _K), jnp.float32),
    ],
    scratch_types=[
        pltpu.VMEM((_N,), jnp.float32),
        pltpu.VMEM((_N,), jnp.float32),
        pltpu.VMEM((_N,), jnp.float32),
        pltpu.VMEM((_N,), jnp.int32),
        pltpu.VMEM((_N,), jnp.int32),
        pltpu.VMEM((_N,), jnp.float32),
        pltpu.VMEM((_N,), jnp.int32),
        pltpu.VMEM((_RPW, _K), jnp.int32),
        pltpu.VMEM((_RPW, _K), jnp.float32),
        pltpu.VMEM((_K,), jnp.int32),
        pltpu.VMEM((_K,), jnp.float32),
    ],
)
def _sc_radius(x_hbm, y_hbm, z_hbm, ss_hbm, se_hbm, nbr_hbm, sq_hbm,
               xv, yv, zv, ssv, sev, cdist, cidx, nbrv, sqv, tnb, tsq):
    wid = lax.axis_index("s") * _NC + lax.axis_index("c")

    pltpu.sync_copy(x_hbm, xv)
    pltpu.sync_copy(y_hbm, yv)
    pltpu.sync_copy(z_hbm, zv)
    pltpu.sync_copy(ss_hbm, ssv)
    pltpu.sync_copy(se_hbm, sev)

    lanes = lax.broadcasted_iota(jnp.int32, (_L,), 0)
    lane0 = lanes == 0
    inf = jnp.float32(jnp.inf)
    inf_vec = jnp.full((_L,), inf, jnp.float32)
    zero_vec = jnp.zeros((_L,), jnp.float32)

    def per_node(t, carry_unused):
        i = t * _NW + wid
        iv = jnp.full((_L,), i, jnp.int32)
        sv = plsc.load_gather(ssv, [iv])
        ev = plsc.load_gather(sev, [iv])
        xiv = plsc.load_gather(xv, [iv])
        yiv = plsc.load_gather(yv, [iv])
        ziv = plsc.load_gather(zv, [iv])
        s = sv[0]
        e = ev[0]
        c0 = s // _L
        c1 = (e + (_L - 1)) // _L

        def dchunk(c, cnt):
            off = pl.multiple_of(c * _L, _L)
            jj = off + lanes
            dx = xiv - xv[pl.ds(off, _L)]
            dy = yiv - yv[pl.ds(off, _L)]
            dz = ziv - zv[pl.ds(off, _L)]
            sq = dx * dx + dy * dy
            sq = sq + dz * dz
            ok = (jj >= sv) & (jj < ev) & (jj != iv) & (sq < _CUT2)
            cs = plsc.cumsum(jnp.where(ok, 1, 0))
            posn = cnt + cs - 1
            plsc.store_scatter(cdist, [posn], sq, mask=ok)
            plsc.store_scatter(cidx, [posn], jj, mask=ok)
            return cnt + cs[_L - 1]

        ncand = lax.fori_loop(c0, c1, dchunk, jnp.int32(0))
        ncv = jnp.broadcast_to(ncand, (_L,))

        nbrv[t, pl.ds(_L, _L)] = iv
        sqv[t, pl.ds(_L, _L)] = zero_vec

        @pl.when(ncand <= _L)
        def _():
            keys = jnp.where(lanes < ncv, cdist[pl.ds(0, _L)], inf)
            vals = cidx[pl.ds(0, _L)]
            ks, vs = plsc.sort_key_val(keys, vals)
            pad = ks == inf
            nbrv[t, pl.ds(0, _L)] = jnp.where(pad, iv, vs)
            sqv[t, pl.ds(0, _L)] = jnp.where(pad, zero_vec, ks)

        @pl.when(ncand > _L)
        def _():
            for q in range(_K // _L):
                tnb[pl.ds(q * _L, _L)] = iv
                tsq[pl.ds(q * _L, _L)] = zero_vec
            nch = (ncand + (_L - 1)) // _L

            def sel_cond(carry):
                k, done = carry
                return jnp.logical_and(k < _K, jnp.logical_not(done))

            def sel_body(carry):
                k, _d = carry

                def mchunk(c, mc):
                    vmin, varg = mc
                    off = pl.multiple_of(c * _L, _L)
                    d = cdist[pl.ds(off, _L)]
                    d = jnp.where(off + lanes < ncv, d, inf)
                    lt = d < vmin
                    return jnp.where(lt, d, vmin), jnp.where(lt, c, varg)

                vmin, varg = lax.fori_loop(
                    0, nch, mchunk,
                    (inf_vec, jnp.zeros((_L,), jnp.int32)))
                minval = jnp.min(vmin)
                found = minval < inf

                @pl.when(found)
                def _():
                    j = jnp.min(jnp.where(vmin == minval, varg * _L + lanes,
                                          jnp.int32(2**30)))
                    jv = jnp.broadcast_to(j, (_L,))
                    nb = plsc.load_gather(cidx, [jv])
                    slotv = jnp.broadcast_to(k, (_L,))
                    plsc.store_scatter(cdist, [jv], inf_vec, mask=lane0)
                    plsc.store_scatter(tnb, [slotv], nb, mask=lane0)
                    plsc.store_scatter(tsq, [slotv],
                                       jnp.broadcast_to(minval, (_L,)),
                                       mask=lane0)

                return k + jnp.where(found, 1, 0), jnp.logical_not(found)

            lax.while_loop(sel_cond, sel_body, (jnp.int32(0), False))
            for q in range(_K // _L):
                nbrv[t, pl.ds(q * _L, _L)] = tnb[pl.ds(q * _L, _L)]
                sqv[t, pl.ds(q * _L, _L)] = tsq[pl.ds(q * _L, _L)]

        return carry_unused

    lax.fori_loop(0, _RPW, per_node, 0)

    pltpu.sync_copy(nbrv, nbr_hbm.at[wid])
    pltpu.sync_copy(sqv, sq_hbm.at[wid])


def _sqrt_body(x_ref, o_ref):
    o_ref[...] = jnp.sqrt(x_ref[...])


_tc_sqrt = pl.pallas_call(
    _sqrt_body,
    out_shape=jax.ShapeDtypeStruct((_N * _K // 1024, 1024), jnp.float32),
)


def _segment_bounds(batch):
    idx = jnp.arange(_N, dtype=jnp.int32)
    ne = batch[1:] != batch[:-1]
    is_start = jnp.concatenate([jnp.ones((1,), bool), ne])
    is_end = jnp.concatenate([ne, jnp.ones((1,), bool)])
    ss = lax.cummax(jnp.where(is_start, idx, 0))
    se = lax.cummin(jnp.where(is_end, idx + 1, _N)[::-1])[::-1]
    return ss, se


def kernel(pos, batch):
    pos = pos.astype(jnp.float32)
    batch = batch.astype(jnp.int32)
    x = pos[:, 0]
    y = pos[:, 1]
    z = pos[:, 2]
    ss, se = _segment_bounds(batch)
    nbr, sq = _sc_radius(x, y, z, ss, se)
    nbr = nbr.transpose(1, 0, 2).reshape(-1)
    sq = sq.transpose(1, 0, 2).reshape(_N * _K // 1024, 1024)
    w = _tc_sqrt(sq).reshape(-1)
    col = jnp.broadcast_to(
        jnp.arange(_N, dtype=jnp.int32)[:, None], (_N, _K)).reshape(-1)
    edge_index = jnp.stack([nbr, col], axis=0)
    return edge_index, w


# speedup vs baseline: 56.4672x; 1.0034x over previous
import functools

import jax
import jax.numpy as jnp
from jax import lax
from jax.experimental import pallas as pl
from jax.experimental.pallas import tpu as pltpu
from jax.experimental.pallas import tpu_sc as plsc

_N = 4096
_K = 32
_CUT2 = 4.0
_L = 16

_info = plsc.get_sparse_core_info()
_NC = _info.num_cores
_NS = _info.num_subcores
_NW = _NC * _NS
_RPW = _N // _NW

_mesh = plsc.VectorSubcoreMesh(core_axis_name="c", subcore_axis_name="s")


@functools.partial(
    pl.kernel,
    mesh=_mesh,
    compiler_params=pltpu.CompilerParams(needs_layout_passes=False),
    out_type=[
        jax.ShapeDtypeStruct((_NW, _RPW, _K), jnp.int32),
        jax.ShapeDtypeStruct((_NW, _RPW, _K), jnp.float32),
    ],
    scratch_types=[
        pltpu.VMEM((_N,), jnp.float32),
        pltpu.VMEM((_N,), jnp.float32),
        pltpu.VMEM((_N,), jnp.float32),
        pltpu.VMEM((_N,), jnp.int32),
        pltpu.VMEM((_N,), jnp.int32),
        pltpu.VMEM((_N + _L,), jnp.float32),
        pltpu.VMEM((_N + _L,), jnp.int32),
        pltpu.VMEM((_RPW, _K), jnp.int32),
        pltpu.VMEM((_RPW, _K), jnp.float32),
        pltpu.VMEM((_K,), jnp.int32),
        pltpu.VMEM((_K,), jnp.float32),
    ],
)
def _sc_radius(x_hbm, y_hbm, z_hbm, ss_hbm, se_hbm, nbr_hbm, sq_hbm,
               xv, yv, zv, ssv, sev, cdist, cidx, nbrv, sqv, tnb, tsq):
    wid = lax.axis_index("s") * _NC + lax.axis_index("c")

    pltpu.sync_copy(x_hbm, xv)
    pltpu.sync_copy(y_hbm, yv)
    pltpu.sync_copy(z_hbm, zv)
    pltpu.sync_copy(ss_hbm, ssv)
    pltpu.sync_copy(se_hbm, sev)

    lanes = lax.broadcasted_iota(jnp.int32, (_L,), 0)
    lane0 = lanes == 0
    inf = jnp.float32(jnp.inf)
    inf_vec = jnp.full((_L,), inf, jnp.float32)
    zero_vec = jnp.zeros((_L,), jnp.float32)

    def per_node(t, carry_unused):
        i = t * _NW + wid
        iv = jnp.full((_L,), i, jnp.int32)
        sv = plsc.load_gather(ssv, [iv])
        ev = plsc.load_gather(sev, [iv])
        xiv = plsc.load_gather(xv, [iv])
        yiv = plsc.load_gather(yv, [iv])
        ziv = plsc.load_gather(zv, [iv])
        s = sv[0]
        e = ev[0]
        c0 = s // _L
        c1 = (e + (_L - 1)) // _L

        def dchunk(c, cnt):
            off = pl.multiple_of(c * _L, _L)
            jj = off + lanes
            dx = xiv - xv[pl.ds(off, _L)]
            dy = yiv - yv[pl.ds(off, _L)]
            dz = ziv - zv[pl.ds(off, _L)]
            sq = dx * dx + dy * dy
            sq = sq + dz * dz
            ok = (jj >= sv) & (jj < ev) & (jj != iv) & (sq < _CUT2)
            plsc.store_compressed(cdist.at[pl.ds(cnt, _L)], sq, mask=ok)
            plsc.store_compressed(cidx.at[pl.ds(cnt, _L)], jj, mask=ok)
            return cnt + plsc.all_reduce_population_count(ok)[0]

        ncand = lax.fori_loop(c0, c1, dchunk, jnp.int32(0))
        ncv = jnp.broadcast_to(ncand, (_L,))

        nbrv[t, pl.ds(_L, _L)] = iv
        sqv[t, pl.ds(_L, _L)] = zero_vec

        keys = jnp.where(lanes < ncv, cdist[pl.ds(0, _L)], inf)
        vals = cidx[pl.ds(0, _L)]
        ks, vs = plsc.sort_key_val(keys, vals)
        pad = ks == inf
        nbrv[t, pl.ds(0, _L)] = jnp.where(pad, iv, vs)
        sqv[t, pl.ds(0, _L)] = jnp.where(pad, zero_vec, ks)

        @pl.when(ncand > _L)
        def _():
            for q in range(_K // _L):
                tnb[pl.ds(q * _L, _L)] = iv
                tsq[pl.ds(q * _L, _L)] = zero_vec
            nch = (ncand + (_L - 1)) // _L

            def sel_cond(carry):
                k, done = carry
                return jnp.logical_and(k < _K, jnp.logical_not(done))

            def sel_body(carry):
                k, _d = carry

                def mchunk(c, mc):
                    vmin, varg = mc
                    off = pl.multiple_of(c * _L, _L)
                    d = cdist[pl.ds(off, _L)]
                    d = jnp.where(off + lanes < ncv, d, inf)
                    lt = d < vmin
                    return jnp.where(lt, d, vmin), jnp.where(lt, c, varg)

                vmin, varg = lax.fori_loop(
                    0, nch, mchunk,
                    (inf_vec, jnp.zeros((_L,), jnp.int32)))
                minval = jnp.min(vmin)
                found = minval < inf

                @pl.when(found)
                def _():
                    j = jnp.min(jnp.where(vmin == minval, varg * _L + lanes,
                                          jnp.int32(2**30)))
                    jv = jnp.broadcast_to(j, (_L,))
                    nb = plsc.load_gather(cidx, [jv])
                    slotv = jnp.broadcast_to(k, (_L,))
                    plsc.store_scatter(cdist, [jv], inf_vec, mask=lane0)
                    plsc.store_scatter(tnb, [slotv], nb, mask=lane0)
                    plsc.store_scatter(tsq, [slotv],
                                       jnp.broadcast_to(minval, (_L,)),
                                       mask=lane0)

                return k + jnp.where(found, 1, 0), jnp.logical_not(found)

            lax.while_loop(sel_cond, sel_body, (jnp.int32(0), False))
            for q in range(_K // _L):
                nbrv[t, pl.ds(q * _L, _L)] = tnb[pl.ds(q * _L, _L)]
                sqv[t, pl.ds(q * _L, _L)] = tsq[pl.ds(q * _L, _L)]

        return carry_unused

    lax.fori_loop(0, _RPW, per_node, 0)

    pltpu.sync_copy(nbrv, nbr_hbm.at[wid])
    pltpu.sync_copy(sqv, sq_hbm.at[wid])


def _sqrt_body(x_ref, o_ref):
    o_ref[...] = jnp.sqrt(x_ref[...])


_tc_sqrt = pl.pallas_call(
    _sqrt_body,
    out_shape=jax.ShapeDtypeStruct((_N * _K // 1024, 1024), jnp.float32),
)


def _segment_bounds(batch):
    idx = jnp.arange(_N, dtype=jnp.int32)
    ne = batch[1:] != batch[:-1]
    is_start = jnp.concatenate([jnp.ones((1,), bool), ne])
    is_end = jnp.concatenate([ne, jnp.ones((1,), bool)])
    ss = lax.cummax(jnp.where(is_start, idx, 0))
    se = lax.cummin(jnp.where(is_end, idx + 1, _N)[::-1])[::-1]
    return ss, se


def kernel(pos, batch):
    pos = pos.astype(jnp.float32)
    batch = batch.astype(jnp.int32)
    x = pos[:, 0]
    y = pos[:, 1]
    z = pos[:, 2]
    ss, se = _segment_bounds(batch)
    nbr, sq = _sc_radius(x, y, z, ss, se)
    nbr = nbr.transpose(1, 0, 2).reshape(-1)
    sq = sq.transpose(1, 0, 2).reshape(_N * _K // 1024, 1024)
    w = _tc_sqrt(sq).reshape(-1)
    col = jnp.broadcast_to(
        jnp.arange(_N, dtype=jnp.int32)[:, None], (_N, _K)).reshape(-1)
    edge_index = jnp.stack([nbr, col], axis=0)
    return edge_index, w


# speedup vs baseline: 58.2319x; 1.0313x over previous
import functools

import jax
import jax.numpy as jnp
from jax import lax
from jax.experimental import pallas as pl
from jax.experimental.pallas import tpu as pltpu
from jax.experimental.pallas import tpu_sc as plsc

_N = 4096
_K = 32
_CUT2 = 4.0
_L = 16

_info = plsc.get_sparse_core_info()
_NC = _info.num_cores
_NS = _info.num_subcores
_NW = _NC * _NS
_RPW = _N // _NW

_mesh = plsc.VectorSubcoreMesh(core_axis_name="c", subcore_axis_name="s")


@functools.partial(
    pl.kernel,
    mesh=_mesh,
    compiler_params=pltpu.CompilerParams(needs_layout_passes=False),
    out_type=[
        jax.ShapeDtypeStruct((_NW, _RPW, _K), jnp.int32),
        jax.ShapeDtypeStruct((_NW, _RPW, _K), jnp.float32),
    ],
    scratch_types=[
        pltpu.VMEM((_N + 4 * _L,), jnp.float32),
        pltpu.VMEM((_N + 4 * _L,), jnp.float32),
        pltpu.VMEM((_N + 4 * _L,), jnp.float32),
        pltpu.VMEM((_N,), jnp.int32),
        pltpu.VMEM((_N,), jnp.int32),
        pltpu.VMEM((_N + _L,), jnp.float32),
        pltpu.VMEM((_N + _L,), jnp.int32),
        pltpu.VMEM((_RPW, _K), jnp.int32),
        pltpu.VMEM((_RPW, _K), jnp.float32),
        pltpu.VMEM((_K,), jnp.int32),
        pltpu.VMEM((_K,), jnp.float32),
    ],
)
def _sc_radius(x_hbm, y_hbm, z_hbm, ss_hbm, se_hbm, nbr_hbm, sq_hbm,
               xv, yv, zv, ssv, sev, cdist, cidx, nbrv, sqv, tnb, tsq):
    wid = lax.axis_index("s") * _NC + lax.axis_index("c")

    pltpu.sync_copy(x_hbm, xv.at[pl.ds(0, _N)])
    pltpu.sync_copy(y_hbm, yv.at[pl.ds(0, _N)])
    pltpu.sync_copy(z_hbm, zv.at[pl.ds(0, _N)])
    pltpu.sync_copy(ss_hbm, ssv)
    pltpu.sync_copy(se_hbm, sev)

    lanes = lax.broadcasted_iota(jnp.int32, (_L,), 0)
    lane0 = lanes == 0
    inf = jnp.float32(jnp.inf)
    inf_vec = jnp.full((_L,), inf, jnp.float32)
    zero_vec = jnp.zeros((_L,), jnp.float32)

    def per_node(t, carry_unused):
        i = t * _NW + wid
        iv = jnp.full((_L,), i, jnp.int32)
        sv = plsc.load_gather(ssv, [iv])
        ev = plsc.load_gather(sev, [iv])
        xiv = plsc.load_gather(xv, [iv])
        yiv = plsc.load_gather(yv, [iv])
        ziv = plsc.load_gather(zv, [iv])
        s = sv[0]
        e = ev[0]
        c0 = s // _L
        c1 = (e + (_L - 1)) // _L

        sqs, jjs, oks = [], [], []
        for q in range(4):
            off = pl.multiple_of(c0 * _L + q * _L, _L)
            jj = off + lanes
            dx = xiv - xv[pl.ds(off, _L)]
            dy = yiv - yv[pl.ds(off, _L)]
            dz = ziv - zv[pl.ds(off, _L)]
            sq = dx * dx + dy * dy
            sq = sq + dz * dz
            ok = (jj >= sv) & (jj < ev) & (jj != iv) & (sq < _CUT2)
            sqs.append(sq)
            jjs.append(jj)
            oks.append(ok)
        pcs = [plsc.all_reduce_population_count(ok)[0] for ok in oks]
        pos = jnp.int32(0)
        for q in range(4):
            plsc.store_compressed(cdist.at[pl.ds(pos, _L)], sqs[q],
                                  mask=oks[q])
            plsc.store_compressed(cidx.at[pl.ds(pos, _L)], jjs[q],
                                  mask=oks[q])
            pos = pos + pcs[q]

        def dchunk(c, cnt):
            off = pl.multiple_of(c * _L, _L)
            jj = off + lanes
            dx = xiv - xv[pl.ds(off, _L)]
            dy = yiv - yv[pl.ds(off, _L)]
            dz = ziv - zv[pl.ds(off, _L)]
            sq = dx * dx + dy * dy
            sq = sq + dz * dz
            ok = (jj >= sv) & (jj < ev) & (jj != iv) & (sq < _CUT2)
            plsc.store_compressed(cdist.at[pl.ds(cnt, _L)], sq, mask=ok)
            plsc.store_compressed(cidx.at[pl.ds(cnt, _L)], jj, mask=ok)
            return cnt + plsc.all_reduce_population_count(ok)[0]

        ncand = lax.fori_loop(c0 + 4, c1, dchunk, pos)
        ncv = jnp.broadcast_to(ncand, (_L,))

        nbrv[t, pl.ds(_L, _L)] = iv
        sqv[t, pl.ds(_L, _L)] = zero_vec

        keys = jnp.where(lanes < ncv, cdist[pl.ds(0, _L)], inf)
        vals = cidx[pl.ds(0, _L)]
        ks, vs = plsc.sort_key_val(keys, vals)
        pad = ks == inf
        nbrv[t, pl.ds(0, _L)] = jnp.where(pad, iv, vs)
        sqv[t, pl.ds(0, _L)] = jnp.where(pad, zero_vec, ks)

        @pl.when(ncand > _L)
        def _():
            for q in range(_K // _L):
                tnb[pl.ds(q * _L, _L)] = iv
                tsq[pl.ds(q * _L, _L)] = zero_vec
            nch = (ncand + (_L - 1)) // _L

            def sel_cond(carry):
                k, done = carry
                return jnp.logical_and(k < _K, jnp.logical_not(done))

            def sel_body(carry):
                k, _d = carry

                def mchunk(c, mc):
                    vmin, varg = mc
                    off = pl.multiple_of(c * _L, _L)
                    d = cdist[pl.ds(off, _L)]
                    d = jnp.where(off + lanes < ncv, d, inf)
                    lt = d < vmin
                    return jnp.where(lt, d, vmin), jnp.where(lt, c, varg)

                vmin, varg = lax.fori_loop(
                    0, nch, mchunk,
                    (inf_vec, jnp.zeros((_L,), jnp.int32)))
                minval = jnp.min(vmin)
                found = minval < inf

                @pl.when(found)
                def _():
                    j = jnp.min(jnp.where(vmin == minval, varg * _L + lanes,
                                          jnp.int32(2**30)))
                    jv = jnp.broadcast_to(j, (_L,))
                    nb = plsc.load_gather(cidx, [jv])
                    slotv = jnp.broadcast_to(k, (_L,))
                    plsc.store_scatter(cdist, [jv], inf_vec, mask=lane0)
                    plsc.store_scatter(tnb, [slotv], nb, mask=lane0)
                    plsc.store_scatter(tsq, [slotv],
                                       jnp.broadcast_to(minval, (_L,)),
                                       mask=lane0)

                return k + jnp.where(found, 1, 0), jnp.logical_not(found)

            lax.while_loop(sel_cond, sel_body, (jnp.int32(0), False))
            for q in range(_K // _L):
                nbrv[t, pl.ds(q * _L, _L)] = tnb[pl.ds(q * _L, _L)]
                sqv[t, pl.ds(q * _L, _L)] = tsq[pl.ds(q * _L, _L)]

        return carry_unused

    lax.fori_loop(0, _RPW, per_node, 0)

    pltpu.sync_copy(nbrv, nbr_hbm.at[wid])
    pltpu.sync_copy(sqv, sq_hbm.at[wid])


def _sqrt_body(x_ref, o_ref):
    o_ref[...] = jnp.sqrt(x_ref[...])


_tc_sqrt = pl.pallas_call(
    _sqrt_body,
    out_shape=jax.ShapeDtypeStruct((_N * _K // 1024, 1024), jnp.float32),
)


def _segment_bounds(batch):
    idx = jnp.arange(_N, dtype=jnp.int32)
    ne = batch[1:] != batch[:-1]
    is_start = jnp.concatenate([jnp.ones((1,), bool), ne])
    is_end = jnp.concatenate([ne, jnp.ones((1,), bool)])
    ss = lax.cummax(jnp.where(is_start, idx, 0))
    se = lax.cummin(jnp.where(is_end, idx + 1, _N)[::-1])[::-1]
    return ss, se


def kernel(pos, batch):
    pos = pos.astype(jnp.float32)
    batch = batch.astype(jnp.int32)
    x = pos[:, 0]
    y = pos[:, 1]
    z = pos[:, 2]
    ss, se = _segment_bounds(batch)
    nbr, sq = _sc_radius(x, y, z, ss, se)
    nbr = nbr.transpose(1, 0, 2).reshape(-1)
    sq = sq.transpose(1, 0, 2).reshape(_N * _K // 1024, 1024)
    w = _tc_sqrt(sq).reshape(-1)
    col = jnp.broadcast_to(
        jnp.arange(_N, dtype=jnp.int32)[:, None], (_N, _K)).reshape(-1)
    edge_index = jnp.stack([nbr, col], axis=0)
    return edge_index, w


# speedup vs baseline: 131.6401x; 2.2606x over previous
import functools

import jax
import jax.numpy as jnp
from jax import lax
from jax.experimental import pallas as pl
from jax.experimental.pallas import tpu as pltpu
from jax.experimental.pallas import tpu_sc as plsc

_N = 4096
_K = 32
_CUT2 = 4.0
_L = 16

_info = plsc.get_sparse_core_info()
_NC = _info.num_cores
_NS = _info.num_subcores
_NW = _NC * _NS
_RPW = _N // _NW

_mesh = plsc.VectorSubcoreMesh(core_axis_name="c", subcore_axis_name="s")


def _rev(v):
    return lax.rev(v, (0,))


def _merge16(ak, av, bk, bv):
    rbk = _rev(bk)
    rbv = _rev(bv)
    p = ak <= rbk
    lk = jnp.minimum(ak, rbk)
    hk = jnp.maximum(ak, rbk)
    lv = jnp.where(p, av, rbv)
    hv = jnp.where(p, rbv, av)
    lk, lv = plsc.sort_key_val(lk, lv)
    hk, hv = plsc.sort_key_val(hk, hv)
    return lk, lv, hk, hv


def _sort_bitonic32(l0k, l0v, l1k, l1v):
    p = l0k <= l1k
    mk = jnp.minimum(l0k, l1k)
    Mk = jnp.maximum(l0k, l1k)
    mv = jnp.where(p, l0v, l1v)
    Mv = jnp.where(p, l1v, l0v)
    mk, mv = plsc.sort_key_val(mk, mv)
    Mk, Mv = plsc.sort_key_val(Mk, Mv)
    return mk, mv, Mk, Mv


def _low32of64(a0k, a0v, a1k, a1v, b0k, b0v, b1k, b1v):
    rb1k = _rev(b1k)
    rb1v = _rev(b1v)
    rb0k = _rev(b0k)
    rb0v = _rev(b0v)
    p0 = a0k <= rb1k
    l0k = jnp.minimum(a0k, rb1k)
    l0v = jnp.where(p0, a0v, rb1v)
    p1 = a1k <= rb0k
    l1k = jnp.minimum(a1k, rb0k)
    l1v = jnp.where(p1, a1v, rb0v)
    return _sort_bitonic32(l0k, l0v, l1k, l1v)


def _low32of48(a0k, a0v, a1k, a1v, bk, bv):
    rbk = _rev(bk)
    rbv = _rev(bv)
    p1 = a1k <= rbk
    l1k = jnp.minimum(a1k, rbk)
    l1v = jnp.where(p1, a1v, rbv)
    return _sort_bitonic32(a0k, a0v, l1k, l1v)


@functools.partial(
    pl.kernel,
    mesh=_mesh,
    compiler_params=pltpu.CompilerParams(needs_layout_passes=False),
    out_type=[
        jax.ShapeDtypeStruct((_NW, _RPW, _K), jnp.int32),
        jax.ShapeDtypeStruct((_NW, _RPW, _K), jnp.float32),
    ],
    scratch_types=[
        pltpu.VMEM((_N + 4 * _L,), jnp.float32),
        pltpu.VMEM((_N + 4 * _L,), jnp.float32),
        pltpu.VMEM((_N + 4 * _L,), jnp.float32),
        pltpu.VMEM((_N,), jnp.int32),
        pltpu.VMEM((_N,), jnp.int32),
        pltpu.VMEM((_RPW, _K), jnp.int32),
        pltpu.VMEM((_RPW, _K), jnp.float32),
    ],
)
def _sc_radius(x_hbm, y_hbm, z_hbm, ss_hbm, se_hbm, nbr_hbm, sq_hbm,
               xv, yv, zv, ssv, sev, nbrv, sqv):
    wid = lax.axis_index("s") * _NC + lax.axis_index("c")

    pltpu.sync_copy(x_hbm, xv.at[pl.ds(0, _N)])
    pltpu.sync_copy(y_hbm, yv.at[pl.ds(0, _N)])
    pltpu.sync_copy(z_hbm, zv.at[pl.ds(0, _N)])
    pltpu.sync_copy(ss_hbm, ssv)
    pltpu.sync_copy(se_hbm, sev)

    lanes = lax.broadcasted_iota(jnp.int32, (_L,), 0)
    inf = jnp.float32(jnp.inf)
    inf_vec = jnp.full((_L,), inf, jnp.float32)
    zero_vec = jnp.zeros((_L,), jnp.float32)
    zero_ivec = jnp.zeros((_L,), jnp.int32)

    def per_node(t, carry_unused):
        i = t * _NW + wid
        iv = jnp.full((_L,), i, jnp.int32)
        sv = plsc.load_gather(ssv, [iv])
        ev = plsc.load_gather(sev, [iv])
        xiv = plsc.load_gather(xv, [iv])
        yiv = plsc.load_gather(yv, [iv])
        ziv = plsc.load_gather(zv, [iv])
        s = sv[0]
        e = ev[0]
        c0 = s // _L
        c1 = (e + (_L - 1)) // _L

        def chunk_keys(off):
            jj = off + lanes
            dx = xiv - xv[pl.ds(off, _L)]
            dy = yiv - yv[pl.ds(off, _L)]
            dz = ziv - zv[pl.ds(off, _L)]
            sq = dx * dx + dy * dy
            sq = sq + dz * dz
            ok = (jj >= sv) & (jj < ev) & (jj != iv) & (sq < _CUT2)
            return jnp.where(ok, sq, inf), jj

        def emit(k0, v0, k1, v1):
            pad0 = k0 == inf
            pad1 = k1 == inf
            nbrv[t, pl.ds(0, _L)] = jnp.where(pad0, iv, v0)
            sqv[t, pl.ds(0, _L)] = jnp.where(pad0, zero_vec, k0)
            nbrv[t, pl.ds(_L, _L)] = jnp.where(pad1, iv, v1)
            sqv[t, pl.ds(_L, _L)] = jnp.where(pad1, zero_vec, k1)

        @pl.when(c1 - c0 <= 4)
        def _():
            ks = []
            vs = []
            for q in range(4):
                off = pl.multiple_of(c0 * _L + q * _L, _L)
                kq, jq = chunk_keys(off)
                kq, jq = plsc.sort_key_val(kq, jq)
                ks.append(kq)
                vs.append(jq)
            a = _merge16(ks[0], vs[0], ks[1], vs[1])
            b = _merge16(ks[2], vs[2], ks[3], vs[3])
            emit(*_low32of64(*a, *b))

        @pl.when(c1 - c0 > 4)
        def _():
            def fold(c, acc):
                off = pl.multiple_of(c * _L, _L)
                kq, jq = chunk_keys(off)
                kq, jq = plsc.sort_key_val(kq, jq)
                return _low32of48(*acc, kq, jq)

            acc0 = (inf_vec, zero_ivec, inf_vec, zero_ivec)
            emit(*lax.fori_loop(c0, c1, fold, acc0))

        return carry_unused

    lax.fori_loop(0, _RPW, per_node, 0)

    pltpu.sync_copy(nbrv, nbr_hbm.at[wid])
    pltpu.sync_copy(sqv, sq_hbm.at[wid])


def _sqrt_body(x_ref, o_ref):
    o_ref[...] = jnp.sqrt(x_ref[...])


_tc_sqrt = pl.pallas_call(
    _sqrt_body,
    out_shape=jax.ShapeDtypeStruct((_N * _K // 1024, 1024), jnp.float32),
)


def _segment_bounds(batch):
    idx = jnp.arange(_N, dtype=jnp.int32)
    ne = batch[1:] != batch[:-1]
    is_start = jnp.concatenate([jnp.ones((1,), bool), ne])
    is_end = jnp.concatenate([ne, jnp.ones((1,), bool)])
    ss = lax.cummax(jnp.where(is_start, idx, 0))
    se = lax.cummin(jnp.where(is_end, idx + 1, _N)[::-1])[::-1]
    return ss, se


def kernel(pos, batch):
    pos = pos.astype(jnp.float32)
    batch = batch.astype(jnp.int32)
    x = pos[:, 0]
    y = pos[:, 1]
    z = pos[:, 2]
    ss, se = _segment_bounds(batch)
    nbr, sq = _sc_radius(x, y, z, ss, se)
    nbr = nbr.transpose(1, 0, 2).reshape(-1)
    sq = sq.transpose(1, 0, 2).reshape(_N * _K // 1024, 1024)
    w = _tc_sqrt(sq).reshape(-1)
    col = jnp.broadcast_to(
        jnp.arange(_N, dtype=jnp.int32)[:, None], (_N, _K)).reshape(-1)
    edge_index = jnp.stack([nbr, col], axis=0)
    return edge_index, w


# speedup vs baseline: 134.5292x; 1.0219x over previous
import functools

import jax
import jax.numpy as jnp
from jax import lax
from jax.experimental import pallas as pl
from jax.experimental.pallas import tpu as pltpu
from jax.experimental.pallas import tpu_sc as plsc

_N = 4096
_K = 32
_CUT2 = 4.0
_L = 16

_info = plsc.get_sparse_core_info()
_NC = _info.num_cores
_NS = _info.num_subcores
_NW = _NC * _NS
_RPW = _N // _NW

_mesh = plsc.VectorSubcoreMesh(core_axis_name="c", subcore_axis_name="s")


def _rev(v):
    return lax.rev(v, (0,))


def _merge16(ak, av, bk, bv):
    rbk = _rev(bk)
    rbv = _rev(bv)
    p = ak <= rbk
    lk = jnp.minimum(ak, rbk)
    hk = jnp.maximum(ak, rbk)
    lv = jnp.where(p, av, rbv)
    hv = jnp.where(p, rbv, av)
    lk, lv = plsc.sort_key_val(lk, lv)
    hk, hv = plsc.sort_key_val(hk, hv)
    return lk, lv, hk, hv


def _sort_bitonic32(l0k, l0v, l1k, l1v):
    p = l0k <= l1k
    mk = jnp.minimum(l0k, l1k)
    Mk = jnp.maximum(l0k, l1k)
    mv = jnp.where(p, l0v, l1v)
    Mv = jnp.where(p, l1v, l0v)
    mk, mv = plsc.sort_key_val(mk, mv)
    Mk, Mv = plsc.sort_key_val(Mk, Mv)
    return mk, mv, Mk, Mv


def _low32of64(a0k, a0v, a1k, a1v, b0k, b0v, b1k, b1v):
    rb1k = _rev(b1k)
    rb1v = _rev(b1v)
    rb0k = _rev(b0k)
    rb0v = _rev(b0v)
    p0 = a0k <= rb1k
    l0k = jnp.minimum(a0k, rb1k)
    l0v = jnp.where(p0, a0v, rb1v)
    p1 = a1k <= rb0k
    l1k = jnp.minimum(a1k, rb0k)
    l1v = jnp.where(p1, a1v, rb0v)
    return _sort_bitonic32(l0k, l0v, l1k, l1v)


def _low32of48(a0k, a0v, a1k, a1v, bk, bv):
    rbk = _rev(bk)
    rbv = _rev(bv)
    p1 = a1k <= rbk
    l1k = jnp.minimum(a1k, rbk)
    l1v = jnp.where(p1, a1v, rbv)
    return _sort_bitonic32(a0k, a0v, l1k, l1v)


@functools.partial(
    pl.kernel,
    mesh=_mesh,
    compiler_params=pltpu.CompilerParams(needs_layout_passes=False),
    out_type=[
        jax.ShapeDtypeStruct((_NW, _RPW, _K), jnp.int32),
        jax.ShapeDtypeStruct((_NW, _RPW, _K), jnp.float32),
    ],
    scratch_types=[
        pltpu.VMEM((_N + 4 * _L,), jnp.float32),
        pltpu.VMEM((_N + 4 * _L,), jnp.float32),
        pltpu.VMEM((_N + 4 * _L,), jnp.float32),
        pltpu.VMEM((_N,), jnp.int32),
        pltpu.VMEM((_RPW,), jnp.int32),
        pltpu.VMEM((_RPW,), jnp.int32),
        pltpu.VMEM((_RPW, _K), jnp.int32),
        pltpu.VMEM((_RPW, _K), jnp.float32),
    ],
)
def _sc_radius(x_hbm, y_hbm, z_hbm, b_hbm, nbr_hbm, sq_hbm,
               xv, yv, zv, bv, ssv, sev, nbrv, sqv):
    wid = lax.axis_index("s") * _NC + lax.axis_index("c")

    pltpu.sync_copy(x_hbm, xv.at[pl.ds(0, _N)])
    pltpu.sync_copy(y_hbm, yv.at[pl.ds(0, _N)])
    pltpu.sync_copy(z_hbm, zv.at[pl.ds(0, _N)])
    pltpu.sync_copy(b_hbm, bv)

    lanes = lax.broadcasted_iota(jnp.int32, (_L,), 0)
    inf = jnp.float32(jnp.inf)
    inf_vec = jnp.full((_L,), inf, jnp.float32)
    zero_vec = jnp.zeros((_L,), jnp.float32)
    zero_ivec = jnp.zeros((_L,), jnp.int32)

    def seg_group(g, carry_unused):
        idx = (g * _L + lanes) * _NW + wid
        val = plsc.load_gather(bv, [idx])

        def bs_step(_, bounds):
            lo_s, hi_s, lo_e, hi_e = bounds
            mid_s = (lo_s + hi_s) // 2
            bm_s = plsc.load_gather(bv, [jnp.minimum(mid_s, _N - 1)])
            p_s = (bm_s < val) & (lo_s < hi_s)
            mid_e = (lo_e + hi_e) // 2
            bm_e = plsc.load_gather(bv, [jnp.minimum(mid_e, _N - 1)])
            p_e = (bm_e <= val) & (lo_e < hi_e)
            return (jnp.where(p_s, mid_s + 1, lo_s),
                    jnp.where(p_s, hi_s, mid_s),
                    jnp.where(p_e, mid_e + 1, lo_e),
                    jnp.where(p_e, hi_e, mid_e))

        z16 = jnp.zeros((_L,), jnp.int32)
        n16 = jnp.full((_L,), _N, jnp.int32)
        lo_s, _hs, lo_e, _he = lax.fori_loop(
            0, 12, bs_step, (z16, n16, z16, n16))
        ssv[pl.ds(g * _L, _L)] = lo_s
        sev[pl.ds(g * _L, _L)] = lo_e
        return carry_unused

    lax.fori_loop(0, _RPW // _L, seg_group, 0)

    def per_node(t, carry_unused):
        i = t * _NW + wid
        iv = jnp.full((_L,), i, jnp.int32)
        tv = jnp.full((_L,), t, jnp.int32)
        sv = plsc.load_gather(ssv, [tv])
        ev = plsc.load_gather(sev, [tv])
        xiv = plsc.load_gather(xv, [iv])
        yiv = plsc.load_gather(yv, [iv])
        ziv = plsc.load_gather(zv, [iv])
        s = sv[0]
        e = ev[0]
        c0 = s // _L
        c1 = (e + (_L - 1)) // _L

        def chunk_keys(off):
            jj = off + lanes
            dx = xiv - xv[pl.ds(off, _L)]
            dy = yiv - yv[pl.ds(off, _L)]
            dz = ziv - zv[pl.ds(off, _L)]
            sq = dx * dx + dy * dy
            sq = sq + dz * dz
            ok = (jj >= sv) & (jj < ev) & (jj != iv) & (sq < _CUT2)
            return jnp.where(ok, sq, inf), jj

        def emit(k0, v0, k1, v1):
            pad0 = k0 == inf
            pad1 = k1 == inf
            nbrv[t, pl.ds(0, _L)] = jnp.where(pad0, iv, v0)
            sqv[t, pl.ds(0, _L)] = jnp.where(pad0, zero_vec, k0)
            nbrv[t, pl.ds(_L, _L)] = jnp.where(pad1, iv, v1)
            sqv[t, pl.ds(_L, _L)] = jnp.where(pad1, zero_vec, k1)

        @pl.when(c1 - c0 <= 4)
        def _():
            ks = []
            vs = []
            for q in range(4):
                off = pl.multiple_of(c0 * _L + q * _L, _L)
                kq, jq = chunk_keys(off)
                kq, jq = plsc.sort_key_val(kq, jq)
                ks.append(kq)
                vs.append(jq)
            a = _merge16(ks[0], vs[0], ks[1], vs[1])
            b = _merge16(ks[2], vs[2], ks[3], vs[3])
            emit(*_low32of64(*a, *b))

        @pl.when(c1 - c0 > 4)
        def _():
            def fold(c, acc):
                off = pl.multiple_of(c * _L, _L)
                kq, jq = chunk_keys(off)
                kq, jq = plsc.sort_key_val(kq, jq)
                return _low32of48(*acc, kq, jq)

            acc0 = (inf_vec, zero_ivec, inf_vec, zero_ivec)
            emit(*lax.fori_loop(c0, c1, fold, acc0))

        return carry_unused

    lax.fori_loop(0, _RPW, per_node, 0)

    pltpu.sync_copy(nbrv, nbr_hbm.at[wid])
    pltpu.sync_copy(sqv, sq_hbm.at[wid])


def _sqrt_body(x_ref, o_ref):
    o_ref[...] = jnp.sqrt(x_ref[...])


_tc_sqrt = pl.pallas_call(
    _sqrt_body,
    out_shape=jax.ShapeDtypeStruct((_N * _K // 1024, 1024), jnp.float32),
)


def kernel(pos, batch):
    pos = pos.astype(jnp.float32)
    batch = batch.astype(jnp.int32)
    x = pos[:, 0]
    y = pos[:, 1]
    z = pos[:, 2]
    nbr, sq = _sc_radius(x, y, z, batch)
    nbr = nbr.transpose(1, 0, 2).reshape(-1)
    sq = sq.transpose(1, 0, 2).reshape(_N * _K // 1024, 1024)
    w = _tc_sqrt(sq).reshape(-1)
    col = jnp.broadcast_to(
        jnp.arange(_N, dtype=jnp.int32)[:, None], (_N, _K)).reshape(-1)
    edge_index = jnp.stack([nbr, col], axis=0)
    return edge_index, w
